# baseline (speedup 1.0000x reference)
import functools

import jax
import jax.numpy as jnp

from jax import lax
from jax.experimental import pallas as pl
from jax.experimental.pallas import tpu as pltpu
from jax.experimental.pallas import tpu_sc as plsc

ID_NUM = 1000000
EMB = 16
B = 16384
L = 50
K = 64

LP = 56
PB = 8
NG = B // PB
ROWS_PER_G = LP * PB
N_ROWS = NG * ROWS_PER_G

NW = 32
G_PER_W = NG // NW
CHUNK_G = 8
CHUNKS_PER_W = G_PER_W // CHUNK_G
CB = CHUNK_G * PB


def _sc_gather(zT, table):
    mesh = plsc.VectorSubcoreMesh(core_axis_name="c", subcore_axis_name="s")

    @functools.partial(
        pl.kernel,
        mesh=mesh,
        out_type=jax.ShapeDtypeStruct((N_ROWS, EMB), jnp.float32),
        scratch_types=[
            pltpu.VMEM((L, CB), jnp.int32),
            pltpu.VMEM((L, CB, EMB), jnp.float32),
            pltpu.VMEM((CHUNK_G * ROWS_PER_G, EMB), jnp.float32),
            pltpu.SemaphoreType.DMA,
        ],
        compiler_params=pltpu.CompilerParams(use_tc_tiling_on_sc=False),
    )
    def k(zT_hbm, table_hbm, out_hbm, idx_v, gbuf, obuf, sem):
        wid = lax.axis_index("s") * 2 + lax.axis_index("c")

        zero = jnp.zeros((EMB,), jnp.float32)
        for g in range(CHUNK_G):
            def zpad(r, carry, g=g):
                obuf[g * ROWS_PER_G + L * PB + r] = zero
                return carry
            lax.fori_loop(0, (LP - L) * PB, zpad, 0)

        def chunk_body(c, carry):
            chunk = wid * CHUNKS_PER_W + c
            b0 = chunk * CB
            pltpu.sync_copy(zT_hbm.at[:, pl.ds(b0, CB)], idx_v)

            def gather_wave(w, carry2):
                copies = []
                for j in range(10):
                    lrow = w * 10 + j
                    copies.append(pltpu.async_copy(
                        table_hbm.at[idx_v.at[lrow]], gbuf.at[lrow], sem))
                for cp in copies:
                    cp.wait()
                return carry2
            lax.fori_loop(0, 5, gather_wave, 0)

            def reorder_g(g, carry3):
                def reorder_l(lrow, carry4):
                    for p in range(PB):
                        obuf[g * ROWS_PER_G + lrow * PB + p] = \
                            gbuf[lrow, g * PB + p]
                    return carry4
                lax.fori_loop(0, L, reorder_l, 0)
                return carry3
            lax.fori_loop(0, CHUNK_G, reorder_g, 0)

            pltpu.sync_copy(
                obuf, out_hbm.at[pl.ds(chunk * CHUNK_G * ROWS_PER_G,
                                       CHUNK_G * ROWS_PER_G)])
            return carry

        lax.fori_loop(0, CHUNKS_PER_W, chunk_body, 0)

    return k(zT, table)


GP = 8


def _seg_mat(rows, cols, seg_dim):
    ii = lax.broadcasted_iota(jnp.int32, (rows, cols), 0)
    jj = lax.broadcasted_iota(jnp.int32, (rows, cols), 1)
    if seg_dim == 0:
        return (ii // K == jj).astype(jnp.float32)
    return (ii == jj // K).astype(jnp.float32)


def _tc_body(ep_ref, u_ref, o_ref):
    ep = ep_ref[...]
    e = jnp.where(ep > 0, ep, jnp.exp(ep) - 1.0)
    ub = u_ref[...].reshape(GP, PB, EMB, K)
    zpad = jnp.zeros((GP, EMB, K), jnp.float32)
    rows = []
    for p in range(PB):
        pieces = [zpad] * p + [ub[:, p]] + [zpad] * (PB - 1 - p)
        rows.append(jnp.concatenate(pieces, axis=2))
    rhs = jnp.concatenate(rows, axis=1)
    out = lax.dot_general(
        e, rhs, (((2,), (1,)), ((0,), (0,))),
        preferred_element_type=jnp.float32)
    m = jnp.max(out, axis=-1, keepdims=True)
    pexp = jnp.exp(out - m)
    seg = _seg_mat(PB * K, PB, 0)
    segT = _seg_mat(PB, PB * K, 1)
    s1 = lax.dot_general(pexp, seg, (((2,), (0,)), ((), ())),
                         preferred_element_type=jnp.float32)
    ssum = lax.dot_general(s1, segT, (((2,), (0,)), ((), ())),
                           preferred_element_type=jnp.float32)
    prob = pexp / ssum
    cols = [prob[:, :L, K * p:K * (p + 1)] for p in range(PB)]
    o_ref[...] = jnp.stack(cols, axis=1).reshape(GP * PB, L, K)


def _tc_compute(ep, u):
    grid = (NG // GP,)
    return pl.pallas_call(
        _tc_body,
        grid=grid,
        in_specs=[
            pl.BlockSpec((GP, LP, PB * EMB), lambda i: (i, 0, 0)),
            pl.BlockSpec((GP * PB, EMB, K), lambda i: (i, 0, 0)),
        ],
        out_specs=pl.BlockSpec((GP * PB, L, K), lambda i: (i, 0, 0)),
        out_shape=jax.ShapeDtypeStruct((B, L, K), jnp.float32),
        compiler_params=pltpu.CompilerParams(
            dimension_semantics=("arbitrary",),
        ),
    )(ep, u)


@jax.jit
def kernel(z, u, table):
    zT = z.T
    ef = _sc_gather(zT, table)
    ep = ef.reshape(NG, LP, PB * EMB)
    return _tc_compute(ep, u)


# speedup vs baseline: 1.0782x; 1.0782x over previous
import functools

import jax
import jax.numpy as jnp

from jax import lax
from jax.experimental import pallas as pl
from jax.experimental.pallas import tpu as pltpu
from jax.experimental.pallas import tpu_sc as plsc

ID_NUM = 1000000
EMB = 16
B = 16384
L = 50
K = 64

LP = 56
PB = 8
NG = B // PB
ROWS_PER_G = LP * PB
N_ROWS = NG * ROWS_PER_G

NW = 32
G_PER_W = NG // NW
CHUNK_G = 8
CHUNKS_PER_W = G_PER_W // CHUNK_G
CB = CHUNK_G * PB


def _sc_gather(zT, table):
    mesh = plsc.VectorSubcoreMesh(core_axis_name="c", subcore_axis_name="s")

    @functools.partial(
        pl.kernel,
        mesh=mesh,
        out_type=jax.ShapeDtypeStruct((N_ROWS, EMB), jnp.float32),
        scratch_types=[
            pltpu.VMEM((L, CB), jnp.int32),
            pltpu.VMEM((L, CB, EMB), jnp.float32),
            pltpu.VMEM((CHUNK_G * ROWS_PER_G, EMB), jnp.float32),
            pltpu.SemaphoreType.DMA,
        ],
        compiler_params=pltpu.CompilerParams(use_tc_tiling_on_sc=False),
    )
    def k(zT_hbm, table_hbm, out_hbm, idx_v, gbuf, obuf, sem):
        wid = lax.axis_index("s") * 2 + lax.axis_index("c")

        zero = jnp.zeros((EMB,), jnp.float32)
        for g in range(CHUNK_G):
            def zpad(r, carry, g=g):
                obuf[g * ROWS_PER_G + L * PB + r] = zero
                return carry
            lax.fori_loop(0, (LP - L) * PB, zpad, 0)

        def chunk_body(c, carry):
            chunk = wid * CHUNKS_PER_W + c
            b0 = chunk * CB
            pltpu.sync_copy(zT_hbm.at[:, pl.ds(b0, CB)], idx_v)

            def gather_wave(w, carry2):
                copies = []
                for j in range(10):
                    lrow = w * 10 + j
                    copies.append(pltpu.async_copy(
                        table_hbm.at[idx_v.at[lrow]], gbuf.at[lrow], sem))
                for cp in copies:
                    cp.wait()
                return carry2
            lax.fori_loop(0, 5, gather_wave, 0)

            def reorder_g(g, carry3):
                def reorder_l(lrow, carry4):
                    for p in range(PB):
                        obuf[g * ROWS_PER_G + lrow * PB + p] = \
                            gbuf[lrow, g * PB + p]
                    return carry4
                lax.fori_loop(0, L, reorder_l, 0)
                return carry3
            lax.fori_loop(0, CHUNK_G, reorder_g, 0)

            pltpu.sync_copy(
                obuf, out_hbm.at[pl.ds(chunk * CHUNK_G * ROWS_PER_G,
                                       CHUNK_G * ROWS_PER_G)])
            return carry

        lax.fori_loop(0, CHUNKS_PER_W, chunk_body, 0)

    return k(zT, table)


GP = 16


def _seg_mat(rows, cols, seg_dim):
    ii = lax.broadcasted_iota(jnp.int32, (rows, cols), 0)
    jj = lax.broadcasted_iota(jnp.int32, (rows, cols), 1)
    if seg_dim == 0:
        return (ii // K == jj).astype(jnp.float32)
    return (ii == jj // K).astype(jnp.float32)


def _tc_body(ep_ref, u_ref, o_ref):
    ep = ep_ref[...]
    e = jnp.where(ep > 0, ep, jnp.exp(ep) - 1.0)
    ub = u_ref[...].reshape(GP, PB, EMB, K)
    zpad = jnp.zeros((GP, EMB, K), jnp.float32)
    rows = []
    for p in range(PB):
        pieces = [zpad] * p + [ub[:, p]] + [zpad] * (PB - 1 - p)
        rows.append(jnp.concatenate(pieces, axis=2))
    rhs = jnp.concatenate(rows, axis=1)
    out = lax.dot_general(
        e, rhs, (((2,), (1,)), ((0,), (0,))),
        preferred_element_type=jnp.float32)
    m = jnp.max(out, axis=-1, keepdims=True)
    pexp = jnp.exp(out - m)
    seg = _seg_mat(PB * K, PB, 0)
    segT = _seg_mat(PB, PB * K, 1)
    s1 = lax.dot_general(pexp, seg, (((2,), (0,)), ((), ())),
                         preferred_element_type=jnp.float32)
    ssum = lax.dot_general(s1, segT, (((2,), (0,)), ((), ())),
                           preferred_element_type=jnp.float32)
    prob = pexp / ssum
    cols = [prob[:, :L, K * p:K * (p + 1)] for p in range(PB)]
    o_ref[...] = jnp.stack(cols, axis=1).reshape(GP * PB, L, K)


def _tc_compute(ep, u):
    grid = (NG // GP,)
    return pl.pallas_call(
        _tc_body,
        grid=grid,
        in_specs=[
            pl.BlockSpec((GP, LP, PB * EMB), lambda i: (i, 0, 0)),
            pl.BlockSpec((GP * PB, EMB, K), lambda i: (i, 0, 0)),
        ],
        out_specs=pl.BlockSpec((GP * PB, L, K), lambda i: (i, 0, 0)),
        out_shape=jax.ShapeDtypeStruct((B, L, K), jnp.float32),
        compiler_params=pltpu.CompilerParams(
            dimension_semantics=("arbitrary",),
        ),
    )(ep, u)


@jax.jit
def kernel(z, u, table):
    zT = z.T
    ef = _sc_gather(zT, table)
    ep = ef.reshape(NG, LP, PB * EMB)
    return _tc_compute(ep, u)


# speedup vs baseline: 1.1196x; 1.0383x over previous
import functools

import jax
import jax.numpy as jnp

from jax import lax
from jax.experimental import pallas as pl
from jax.experimental.pallas import tpu as pltpu
from jax.experimental.pallas import tpu_sc as plsc

ID_NUM = 1000000
EMB = 16
B = 16384
L = 50
K = 64

LP = 56
PB = 8
NG = B // PB
ROWS_PER_G = LP * PB
N_ROWS = NG * ROWS_PER_G

NW = 32
G_PER_W = NG // NW
CHUNK_G = 8
CHUNKS_PER_W = G_PER_W // CHUNK_G
CB = CHUNK_G * PB


def _sc_gather(zT, table):
    mesh = plsc.VectorSubcoreMesh(core_axis_name="c", subcore_axis_name="s")

    @functools.partial(
        pl.kernel,
        mesh=mesh,
        out_type=jax.ShapeDtypeStruct((N_ROWS, EMB), jnp.float32),
        scratch_types=[
            pltpu.VMEM((L, CB), jnp.int32),
            pltpu.VMEM((L, CB, EMB), jnp.float32),
            pltpu.VMEM((CHUNK_G * ROWS_PER_G, EMB), jnp.float32),
            pltpu.SemaphoreType.DMA,
        ],
        compiler_params=pltpu.CompilerParams(use_tc_tiling_on_sc=False),
    )
    def k(zT_hbm, table_hbm, out_hbm, idx_v, gbuf, obuf, sem):
        wid = lax.axis_index("s") * 2 + lax.axis_index("c")

        zero = jnp.zeros((EMB,), jnp.float32)
        for g in range(CHUNK_G):
            def zpad(r, carry, g=g):
                obuf[g * ROWS_PER_G + L * PB + r] = zero
                return carry
            lax.fori_loop(0, (LP - L) * PB, zpad, 0)

        def chunk_body(c, carry):
            chunk = wid * CHUNKS_PER_W + c
            b0 = chunk * CB
            pltpu.sync_copy(zT_hbm.at[:, pl.ds(b0, CB)], idx_v)

            def gather_wave(w, carry2):
                copies = []
                for j in range(10):
                    lrow = w * 10 + j
                    copies.append(pltpu.async_copy(
                        table_hbm.at[idx_v.at[lrow]], gbuf.at[lrow], sem))
                for cp in copies:
                    cp.wait()
                return carry2
            lax.fori_loop(0, 5, gather_wave, 0)

            def reorder_g(g, carry3):
                def reorder_l(lrow, carry4):
                    for p in range(PB):
                        obuf[g * ROWS_PER_G + lrow * PB + p] = \
                            gbuf[lrow, g * PB + p]
                    return carry4
                lax.fori_loop(0, L, reorder_l, 0)
                return carry3
            lax.fori_loop(0, CHUNK_G, reorder_g, 0)

            pltpu.sync_copy(
                obuf, out_hbm.at[pl.ds(chunk * CHUNK_G * ROWS_PER_G,
                                       CHUNK_G * ROWS_PER_G)])
            return carry

        lax.fori_loop(0, CHUNKS_PER_W, chunk_body, 0)

    return k(zT, table)


GP = 32


def _seg_mat(rows, cols, seg_dim):
    ii = lax.broadcasted_iota(jnp.int32, (rows, cols), 0)
    jj = lax.broadcasted_iota(jnp.int32, (rows, cols), 1)
    if seg_dim == 0:
        return (ii // K == jj).astype(jnp.float32)
    return (ii == jj // K).astype(jnp.float32)


def _tc_body(ep_ref, u_ref, o_ref):
    ep = ep_ref[...]
    e = jnp.where(ep > 0, ep, jnp.exp(ep) - 1.0)
    ub = u_ref[...].reshape(GP, PB, EMB, K)
    zpad = jnp.zeros((GP, EMB, K), jnp.float32)
    rows = []
    for p in range(PB):
        pieces = [zpad] * p + [ub[:, p]] + [zpad] * (PB - 1 - p)
        rows.append(jnp.concatenate(pieces, axis=2))
    rhs = jnp.concatenate(rows, axis=1)
    out = lax.dot_general(
        e, rhs, (((2,), (1,)), ((0,), (0,))),
        preferred_element_type=jnp.float32)
    m = jnp.max(out, axis=-1, keepdims=True)
    pexp = jnp.exp(out - m)
    seg = _seg_mat(PB * K, PB, 0)
    segT = _seg_mat(PB, PB * K, 1)
    s1 = lax.dot_general(pexp, seg, (((2,), (0,)), ((), ())),
                         preferred_element_type=jnp.float32)
    ssum = lax.dot_general(s1, segT, (((2,), (0,)), ((), ())),
                           preferred_element_type=jnp.float32)
    prob = pexp / ssum
    cols = [prob[:, :L, K * p:K * (p + 1)] for p in range(PB)]
    o_ref[...] = jnp.stack(cols, axis=1).reshape(GP * PB, L, K)


def _tc_compute(ep, u):
    grid = (NG // GP,)
    return pl.pallas_call(
        _tc_body,
        grid=grid,
        in_specs=[
            pl.BlockSpec((GP, LP, PB * EMB), lambda i: (i, 0, 0)),
            pl.BlockSpec((GP * PB, EMB, K), lambda i: (i, 0, 0)),
        ],
        out_specs=pl.BlockSpec((GP * PB, L, K), lambda i: (i, 0, 0)),
        out_shape=jax.ShapeDtypeStruct((B, L, K), jnp.float32),
        compiler_params=pltpu.CompilerParams(
            dimension_semantics=("arbitrary",),
        ),
    )(ep, u)


@jax.jit
def kernel(z, u, table):
    zT = z.T
    ef = _sc_gather(zT, table)
    ep = ef.reshape(NG, LP, PB * EMB)
    return _tc_compute(ep, u)


# speedup vs baseline: 1.1275x; 1.0071x over previous
import functools

import jax
import jax.numpy as jnp

from jax import lax
from jax.experimental import pallas as pl
from jax.experimental.pallas import tpu as pltpu
from jax.experimental.pallas import tpu_sc as plsc

ID_NUM = 1000000
EMB = 16
B = 16384
L = 50
K = 64

LP = 56
PB = 8
NG = B // PB
ROWS_PER_G = LP * PB
N_ROWS = NG * ROWS_PER_G

NW = 32
G_PER_W = NG // NW
CHUNK_G = 8
CHUNKS_PER_W = G_PER_W // CHUNK_G
CB = CHUNK_G * PB


def _sc_gather(zT, table):
    mesh = plsc.VectorSubcoreMesh(core_axis_name="c", subcore_axis_name="s")

    @functools.partial(
        pl.kernel,
        mesh=mesh,
        out_type=jax.ShapeDtypeStruct((N_ROWS, EMB), jnp.float32),
        scratch_types=[
            pltpu.VMEM((L, CB), jnp.int32),
            pltpu.VMEM((L, CB, EMB), jnp.float32),
            pltpu.VMEM((CHUNK_G * ROWS_PER_G, EMB), jnp.float32),
            pltpu.SemaphoreType.DMA,
        ],
        compiler_params=pltpu.CompilerParams(use_tc_tiling_on_sc=False),
    )
    def k(zT_hbm, table_hbm, out_hbm, idx_v, gbuf, obuf, sem):
        wid = lax.axis_index("s") * 2 + lax.axis_index("c")

        zero = jnp.zeros((EMB,), jnp.float32)
        for g in range(CHUNK_G):
            def zpad(r, carry, g=g):
                obuf[g * ROWS_PER_G + L * PB + r] = zero
                return carry
            lax.fori_loop(0, (LP - L) * PB, zpad, 0)

        def chunk_body(c, carry):
            chunk = wid * CHUNKS_PER_W + c
            b0 = chunk * CB
            pltpu.sync_copy(zT_hbm.at[:, pl.ds(b0, CB)], idx_v)

            def gather_wave(w, carry2):
                copies = []
                for j in range(10):
                    lrow = w * 10 + j
                    copies.append(pltpu.async_copy(
                        table_hbm.at[idx_v.at[lrow]], gbuf.at[lrow], sem))
                for cp in copies:
                    cp.wait()
                return carry2
            lax.fori_loop(0, 5, gather_wave, 0)

            def reorder_g(g, carry3):
                def reorder_l(lrow, carry4):
                    for p in range(PB):
                        obuf[g * ROWS_PER_G + lrow * PB + p] = \
                            gbuf[lrow, g * PB + p]
                    return carry4
                lax.fori_loop(0, L, reorder_l, 0)
                return carry3
            lax.fori_loop(0, CHUNK_G, reorder_g, 0)

            pltpu.sync_copy(
                obuf, out_hbm.at[pl.ds(chunk * CHUNK_G * ROWS_PER_G,
                                       CHUNK_G * ROWS_PER_G)])
            return carry

        lax.fori_loop(0, CHUNKS_PER_W, chunk_body, 0)

    return k(zT, table)


GP = 64


def _seg_mat(rows, cols, seg_dim):
    ii = lax.broadcasted_iota(jnp.int32, (rows, cols), 0)
    jj = lax.broadcasted_iota(jnp.int32, (rows, cols), 1)
    if seg_dim == 0:
        return (ii // K == jj).astype(jnp.float32)
    return (ii == jj // K).astype(jnp.float32)


def _tc_body(ep_ref, u_ref, o_ref):
    ep = ep_ref[...]
    e = jnp.where(ep > 0, ep, jnp.exp(ep) - 1.0)
    ub = u_ref[...].reshape(GP, PB, EMB, K)
    zpad = jnp.zeros((GP, EMB, K), jnp.float32)
    rows = []
    for p in range(PB):
        pieces = [zpad] * p + [ub[:, p]] + [zpad] * (PB - 1 - p)
        rows.append(jnp.concatenate(pieces, axis=2))
    rhs = jnp.concatenate(rows, axis=1)
    out = lax.dot_general(
        e, rhs, (((2,), (1,)), ((0,), (0,))),
        preferred_element_type=jnp.float32)
    m = jnp.max(out, axis=-1, keepdims=True)
    pexp = jnp.exp(out - m)
    seg = _seg_mat(PB * K, PB, 0)
    segT = _seg_mat(PB, PB * K, 1)
    s1 = lax.dot_general(pexp, seg, (((2,), (0,)), ((), ())),
                         preferred_element_type=jnp.float32)
    ssum = lax.dot_general(s1, segT, (((2,), (0,)), ((), ())),
                           preferred_element_type=jnp.float32)
    prob = pexp / ssum
    cols = [prob[:, :L, K * p:K * (p + 1)] for p in range(PB)]
    o_ref[...] = jnp.stack(cols, axis=1).reshape(GP * PB, L, K)


def _tc_compute(ep, u):
    grid = (NG // GP,)
    return pl.pallas_call(
        _tc_body,
        grid=grid,
        in_specs=[
            pl.BlockSpec((GP, LP, PB * EMB), lambda i: (i, 0, 0)),
            pl.BlockSpec((GP * PB, EMB, K), lambda i: (i, 0, 0)),
        ],
        out_specs=pl.BlockSpec((GP * PB, L, K), lambda i: (i, 0, 0)),
        out_shape=jax.ShapeDtypeStruct((B, L, K), jnp.float32),
        compiler_params=pltpu.CompilerParams(
            dimension_semantics=("arbitrary",),
        ),
    )(ep, u)


@jax.jit
def kernel(z, u, table):
    zT = z.T
    ef = _sc_gather(zT, table)
    ep = ef.reshape(NG, LP, PB * EMB)
    return _tc_compute(ep, u)
